# baseline (speedup 1.0000x reference)
import functools

import jax
import jax.numpy as jnp
from jax import lax
from jax.experimental import pallas as pl
from jax.experimental.pallas import tpu as pltpu
from jax.experimental.pallas import tpu_sc as plsc

B = 4096
L = 200
D = 32
NC = 2
NS = 16
NW = NC * NS

BH = B * L
ROWS_B = B // NW
HPW = ROWS_B * L
LT = L // 8
K = 4
CR = K * 128
NCHUNK = HPW // CR
NBODY = NCHUNK // 2


def _body(uid, iid, cid, hid, hcd, w_user, w_item, w_cate,
          user_out, item_out, hist_out,
          idx_b, rows_b, cb, fi, fc, rows_i, rows_c,
          sem_b, sem_gi0, sem_gi1, sem_gc0, sem_gc1,
          sem_wi0, sem_wi1, sem_wc0, sem_wc1):
    wid = lax.axis_index("s") * NC + lax.axis_index("c")

    base = wid * ROWS_B

    def small_lookup(ids2d, table, out_ref, col):
        pltpu.sync_copy(ids2d.at[pl.ds(wid, 1)], idx_b)
        pltpu.async_copy(table.at[idx_b.at[0]], rows_b, sem_b).wait()
        pltpu.sync_copy(rows_b, out_ref.at[pl.ds(base, ROWS_B), pl.ds(col, D)])

    small_lookup(uid, w_user, user_out, 0)
    small_lookup(iid, w_item, item_out, 0)
    small_lookup(cid, w_cate, item_out, D)

    lane = jnp.arange(16, dtype=jnp.int32) * L

    def build_flat(ids2d, flat):
        def tile_body(lt, carry):
            pltpu.sync_copy(
                ids2d.at[pl.ds(lt * 8, 8), pl.ds(wid * 128, 128)], cb)
            for s in range(8):
                for g in range(8):
                    v = cb[s, pl.ds(16 * g, 16)]
                    pos = lane + (g * 16 * L + lt * 8 + s)
                    plsc.store_scatter(flat, [pos], v)
            return carry

        lax.fori_loop(0, LT, tile_body, 0)

    build_flat(hid, fi)
    build_flat(hcd, fc)

    hrow0 = wid * HPW

    def fires(table, flat, slot, c, rows_ref, sem):
        return [
            pltpu.async_copy(
                table.at[flat.at[pl.ds(c * CR + j * 128, 128)]],
                rows_ref.at[pl.ds((slot * K + j) * 128, 128)], sem)
            for j in range(K)
        ]

    def write_cp(rows_ref, slot, c, col, sem):
        return pltpu.make_async_copy(
            rows_ref.at[pl.ds(slot * CR, CR)],
            hist_out.at[pl.ds(hrow0 + c * CR, CR), pl.ds(col, D)],
            sem)

    def body(g, carry):
        c0 = 2 * g
        c1 = 2 * g + 1
        @pl.when(g > 0)
        def _():
            write_cp(rows_i, 0, c0 - 2, 0, sem_wi0).wait()
        gi0 = fires(w_item, fi, 0, c0, rows_i, sem_gi0)

        @pl.when(g > 0)
        def _():
            write_cp(rows_c, 0, c0 - 2, D, sem_wc0).wait()
        gc0 = fires(w_cate, fc, 0, c0, rows_c, sem_gc0)

        @pl.when(g > 0)
        def _():
            write_cp(rows_i, 1, c1 - 2, 0, sem_wi1).wait()
        gi1 = fires(w_item, fi, 1, c1, rows_i, sem_gi1)

        @pl.when(g > 0)
        def _():
            write_cp(rows_c, 1, c1 - 2, D, sem_wc1).wait()
        gc1 = fires(w_cate, fc, 1, c1, rows_c, sem_gc1)

        for cp in gi0:
            cp.wait()
        write_cp(rows_i, 0, c0, 0, sem_wi0).start()
        for cp in gc0:
            cp.wait()
        write_cp(rows_c, 0, c0, D, sem_wc0).start()
        for cp in gi1:
            cp.wait()
        write_cp(rows_i, 1, c1, 0, sem_wi1).start()
        for cp in gc1:
            cp.wait()
        write_cp(rows_c, 1, c1, D, sem_wc1).start()
        return carry

    lax.fori_loop(0, NBODY, body, 0)

    write_cp(rows_i, 0, NCHUNK - 2, 0, sem_wi0).wait()
    write_cp(rows_c, 0, NCHUNK - 2, D, sem_wc0).wait()
    write_cp(rows_i, 1, NCHUNK - 1, 0, sem_wi1).wait()
    write_cp(rows_c, 1, NCHUNK - 1, D, sem_wc1).wait()


@jax.jit
def _run(uid, iid, cid, hid, hcd, w_user, w_item, w_cate):
    kern = pl.kernel(
        _body,
        out_type=[
            jax.ShapeDtypeStruct((B, D), jnp.float32),
            jax.ShapeDtypeStruct((B, 2 * D), jnp.float32),
            jax.ShapeDtypeStruct((BH, 2 * D), jnp.float32),
        ],
        mesh=plsc.VectorSubcoreMesh(core_axis_name="c", subcore_axis_name="s"),
        compiler_params=pltpu.CompilerParams(
            use_tc_tiling_on_sc=False, needs_layout_passes=False),
        scratch_types=[
            pltpu.VMEM((1, 128), jnp.int32),
            pltpu.VMEM((128, D), jnp.float32),
            pltpu.VMEM((8, 128), jnp.int32),
            pltpu.VMEM((HPW,), jnp.int32),
            pltpu.VMEM((HPW,), jnp.int32),
            pltpu.VMEM((2 * CR, D), jnp.float32),
            pltpu.VMEM((2 * CR, D), jnp.float32),
        ] + [pltpu.SemaphoreType.DMA] * 9,
    )
    return kern(uid, iid, cid, hid, hcd, w_user, w_item, w_cate)


def kernel(user_id, item_id, cate_id, hist_item_id, hist_cate_id,
           W_user_id, W_item_id, W_cate_id):
    uid = user_id.astype(jnp.int32).reshape(NW, ROWS_B)
    iid = item_id.astype(jnp.int32).reshape(NW, ROWS_B)
    cid = cate_id.astype(jnp.int32).reshape(NW, ROWS_B)
    hid = hist_item_id.astype(jnp.int32).T
    hcd = hist_cate_id.astype(jnp.int32).T
    user_emb, item_emb, hist_flat = _run(
        uid, iid, cid, hid, hcd, W_user_id, W_item_id, W_cate_id)
    return user_emb, item_emb, hist_flat.reshape(B, L, 2 * D)
